# baseline (speedup 1.0000x reference)
import functools

import jax
import jax.numpy as jnp
from jax import lax
from jax.experimental import pallas as pl
from jax.experimental.pallas import tpu as pltpu
from jax.experimental.pallas import tpu_sc as plsc

D_MODEL = 1024
NUM = 16384

NUM_CORES = 2
NUM_SUBCORES = 16
NUM_WORKERS = NUM_CORES * NUM_SUBCORES

B_PER_W = NUM // NUM_WORKERS
CHUNK = 32
NCHUNKS = B_PER_W // CHUNK


def _gather_body(table_hbm, idx_hbm, out_hbm, idx_v, rows_v, semg0, semg1,
                 semw0, semw1):
    wid = lax.axis_index("s") * NUM_CORES + lax.axis_index("c")
    base = wid * B_PER_W
    semg = (semg0, semg1)
    semw = (semw0, semw1)

    pltpu.sync_copy(idx_hbm.at[pl.ds(wid * NCHUNKS, NCHUNKS)], idx_v)

    def start_gather(g, b):
        pltpu.async_copy(table_hbm.at[idx_v.at[g]], rows_v.at[b], semg[b])

    def wait_gather(b):
        pltpu.make_async_copy(
            table_hbm.at[pl.ds(0, CHUNK)], rows_v.at[b], semg[b]
        ).wait()

    def start_writeback(g, b):
        pltpu.async_copy(
            rows_v.at[b], out_hbm.at[pl.ds(base + g * CHUNK, CHUNK)], semw[b]
        )

    def wait_writeback(b):
        pltpu.make_async_copy(
            rows_v.at[b], out_hbm.at[pl.ds(base, CHUNK)], semw[b]
        ).wait()

    start_gather(0, 0)
    for i in range(NCHUNKS):
        b = i % 2
        wait_gather(b)
        start_writeback(i, b)
        if i + 1 < NCHUNKS:
            if i >= 1:
                wait_writeback(1 - b)
            start_gather(i + 1, 1 - b)
    wait_writeback(0)
    wait_writeback(1)


@jax.jit
def _positional_gather(encoding, idx):
    kernel_fn = pl.kernel(
        _gather_body,
        out_type=jax.ShapeDtypeStruct((NUM, D_MODEL), jnp.float32),
        mesh=plsc.VectorSubcoreMesh(core_axis_name="c", subcore_axis_name="s"),
        scratch_types=[
            pltpu.VMEM((NCHUNKS, CHUNK), jnp.int32),
            pltpu.VMEM((2, CHUNK, D_MODEL), jnp.float32),
            pltpu.SemaphoreType.DMA,
            pltpu.SemaphoreType.DMA,
            pltpu.SemaphoreType.DMA,
            pltpu.SemaphoreType.DMA,
        ],
    )
    return kernel_fn(encoding, idx)


def kernel(encoding, t):
    idx = t.reshape(NUM).astype(jnp.int32).reshape(NUM // CHUNK, CHUNK)
    return _positional_gather(encoding, idx)


# speedup vs baseline: 1.0320x; 1.0320x over previous
import functools

import jax
import jax.numpy as jnp
from jax import lax
from jax.experimental import pallas as pl
from jax.experimental.pallas import tpu as pltpu
from jax.experimental.pallas import tpu_sc as plsc

D_MODEL = 1024
NUM = 16384

NUM_CORES = 2
NUM_SUBCORES = 16
NUM_WORKERS = NUM_CORES * NUM_SUBCORES

B_PER_W = NUM // NUM_WORKERS
CHUNK = 32
NCHUNKS = B_PER_W // CHUNK


NBUF = 3


def _gather_body(table_hbm, idx_hbm, out_hbm, idx_v, rows_v, semg0, semg1,
                 semg2, semw0, semw1, semw2):
    wid = lax.axis_index("s") * NUM_CORES + lax.axis_index("c")
    base = wid * B_PER_W
    semg = (semg0, semg1, semg2)
    semw = (semw0, semw1, semw2)

    pltpu.sync_copy(idx_hbm.at[pl.ds(wid * NCHUNKS, NCHUNKS)], idx_v)

    def start_gather(g, b):
        pltpu.async_copy(table_hbm.at[idx_v.at[g]], rows_v.at[b], semg[b])

    def wait_gather(b):
        pltpu.make_async_copy(
            table_hbm.at[pl.ds(0, CHUNK)], rows_v.at[b], semg[b]
        ).wait()

    def start_writeback(g, b):
        pltpu.async_copy(
            rows_v.at[b], out_hbm.at[pl.ds(base + g * CHUNK, CHUNK)], semw[b]
        )

    def wait_writeback(b):
        pltpu.make_async_copy(
            rows_v.at[b], out_hbm.at[pl.ds(base, CHUNK)], semw[b]
        ).wait()

    start_gather(0, 0)
    start_gather(1, 1)
    for i in range(NCHUNKS):
        b = i % NBUF
        wait_gather(b)
        start_writeback(i, b)
        g_next = i + 2
        if g_next < NCHUNKS:
            bn = g_next % NBUF
            if i >= 1:
                wait_writeback(bn)
            start_gather(g_next, bn)
    for g in (NCHUNKS - 3, NCHUNKS - 2, NCHUNKS - 1):
        wait_writeback(g % NBUF)


@jax.jit
def _positional_gather(encoding, idx):
    kernel_fn = pl.kernel(
        _gather_body,
        out_type=jax.ShapeDtypeStruct((NUM, D_MODEL), jnp.float32),
        mesh=plsc.VectorSubcoreMesh(core_axis_name="c", subcore_axis_name="s"),
        scratch_types=[
            pltpu.VMEM((NCHUNKS, CHUNK), jnp.int32),
            pltpu.VMEM((NBUF, CHUNK, D_MODEL), jnp.float32),
            pltpu.SemaphoreType.DMA,
            pltpu.SemaphoreType.DMA,
            pltpu.SemaphoreType.DMA,
            pltpu.SemaphoreType.DMA,
            pltpu.SemaphoreType.DMA,
            pltpu.SemaphoreType.DMA,
        ],
    )
    return kernel_fn(encoding, idx)


def kernel(encoding, t):
    idx = t.reshape(NUM).astype(jnp.int32).reshape(NUM // CHUNK, CHUNK)
    return _positional_gather(encoding, idx)
